# baseline (speedup 1.0000x reference)
import functools

import jax
import jax.numpy as jnp
from jax import lax
from jax.experimental import pallas as pl
from jax.experimental.pallas import tpu as pltpu
from jax.experimental.pallas import tpu_sc as plsc

_info = plsc.get_sparse_core_info()
_NC = _info.num_cores

_ROWS_PER_W = 8


@functools.partial(jax.jit, static_argnums=(2, 3))
def _sc_embedding_lookup(table, idx_pad, pad_rows, hidden):
    num_w = pad_rows // _ROWS_PER_W
    mesh = plsc.VectorSubcoreMesh(core_axis_name="c", subcore_axis_name="s")

    @functools.partial(
        pl.kernel,
        mesh=mesh,
        out_type=jax.ShapeDtypeStruct((pad_rows, hidden), jnp.float32),
        scratch_types=[
            pltpu.VMEM((_ROWS_PER_W,), jnp.int32),
            pltpu.VMEM((_ROWS_PER_W, hidden), jnp.float32),
            pltpu.SemaphoreType.DMA,
        ],
    )
    def gather_kernel(table_hbm, idx_hbm, out_hbm, idx_v, rows_v, sem):
        wid = lax.axis_index("s") * _NC + lax.axis_index("c")

        @pl.when(wid < num_w)
        def _():
            base = wid * _ROWS_PER_W
            pltpu.sync_copy(idx_hbm.at[pl.ds(base, _ROWS_PER_W)], idx_v)
            pltpu.async_copy(table_hbm.at[idx_v], rows_v, sem).wait()
            pltpu.sync_copy(rows_v, out_hbm.at[pl.ds(base, _ROWS_PER_W)])

    return gather_kernel(table, idx_pad)


def kernel(embedding_weight, seq_indices):
    rows = seq_indices.shape[0]
    hidden = embedding_weight.shape[1]
    pad_rows = (rows + _ROWS_PER_W - 1) // _ROWS_PER_W * _ROWS_PER_W
    idx = jnp.asarray(seq_indices, jnp.int32)
    idx_pad = jnp.pad(idx, (0, pad_rows - rows))
    out = _sc_embedding_lookup(embedding_weight, idx_pad, pad_rows, hidden)
    return out[:rows]


# speedup vs baseline: 1.0098x; 1.0098x over previous
import functools

import jax
import jax.numpy as jnp
from jax import lax
from jax.experimental import pallas as pl
from jax.experimental.pallas import tpu as pltpu
from jax.experimental.pallas import tpu_sc as plsc

_info = plsc.get_sparse_core_info()
_NC = _info.num_cores

_ROWS_PER_W = 8


@functools.partial(jax.jit, static_argnums=(2,))
def _sc_embedding_lookup(table, idx, rows):
    hidden = table.shape[1]
    num_full = rows // _ROWS_PER_W
    tail = rows % _ROWS_PER_W
    mesh = plsc.VectorSubcoreMesh(core_axis_name="c", subcore_axis_name="s")

    scratch = [
        pltpu.VMEM((_ROWS_PER_W,), jnp.int32),
        pltpu.VMEM((_ROWS_PER_W, hidden), jnp.float32),
        pltpu.SemaphoreType.DMA,
    ]
    if tail:
        scratch.append(pltpu.VMEM((tail,), jnp.int32))
        scratch.append(pltpu.VMEM((tail, hidden), jnp.float32))

    @functools.partial(
        pl.kernel,
        mesh=mesh,
        out_type=jax.ShapeDtypeStruct((rows, hidden), jnp.float32),
        scratch_types=scratch,
    )
    def gather_kernel(table_hbm, idx_hbm, out_hbm, idx_v, rows_v, sem, *maybe_tail):
        wid = lax.axis_index("s") * _NC + lax.axis_index("c")

        @pl.when(wid < num_full)
        def _():
            base = wid * _ROWS_PER_W
            pltpu.sync_copy(idx_hbm.at[pl.ds(base, _ROWS_PER_W)], idx_v)
            pltpu.async_copy(table_hbm.at[idx_v], rows_v, sem).wait()
            pltpu.sync_copy(rows_v, out_hbm.at[pl.ds(base, _ROWS_PER_W)])

        if tail:
            idx_t, rows_t = maybe_tail

            @pl.when(wid == num_full)
            def _():
                base = num_full * _ROWS_PER_W
                pltpu.sync_copy(idx_hbm.at[pl.ds(base, tail)], idx_t)
                pltpu.async_copy(table_hbm.at[idx_t], rows_t, sem).wait()
                pltpu.sync_copy(rows_t, out_hbm.at[pl.ds(base, tail)])

    return gather_kernel(table, idx)


def kernel(embedding_weight, seq_indices):
    rows = seq_indices.shape[0]
    idx = jnp.asarray(seq_indices, jnp.int32)
    return _sc_embedding_lookup(embedding_weight, idx, rows)


# speedup vs baseline: 1.0679x; 1.0575x over previous
import functools

import jax
import jax.numpy as jnp
from jax import lax
from jax.experimental import pallas as pl
from jax.experimental.pallas import tpu as pltpu
from jax.experimental.pallas import tpu_sc as plsc

_info = plsc.get_sparse_core_info()
_NC = _info.num_cores

_ROWS_PER_W = 8


@functools.partial(jax.jit, static_argnums=(2,))
def _sc_embedding_lookup(table, idx, rows):
    hidden = table.shape[1]
    num_full = rows // _ROWS_PER_W
    tail = rows % _ROWS_PER_W
    mesh = plsc.VectorSubcoreMesh(
        core_axis_name="c", subcore_axis_name="s", num_cores=1
    )

    scratch = [
        pltpu.VMEM((_ROWS_PER_W,), jnp.int32),
        pltpu.VMEM((_ROWS_PER_W, hidden), jnp.float32),
        pltpu.SemaphoreType.DMA,
    ]
    if tail:
        scratch.append(pltpu.VMEM((tail,), jnp.int32))
        scratch.append(pltpu.VMEM((tail, hidden), jnp.float32))

    @functools.partial(
        pl.kernel,
        mesh=mesh,
        out_type=jax.ShapeDtypeStruct((rows, hidden), jnp.float32),
        scratch_types=scratch,
    )
    def gather_kernel(table_hbm, idx_hbm, out_hbm, idx_v, rows_v, sem, *maybe_tail):
        wid = lax.axis_index("s")

        @pl.when(wid < num_full)
        def _():
            base = wid * _ROWS_PER_W
            pltpu.sync_copy(idx_hbm.at[pl.ds(base, _ROWS_PER_W)], idx_v)
            pltpu.async_copy(table_hbm.at[idx_v], rows_v, sem).wait()
            pltpu.sync_copy(rows_v, out_hbm.at[pl.ds(base, _ROWS_PER_W)])

        if tail:
            idx_t, rows_t = maybe_tail

            @pl.when(wid == num_full)
            def _():
                base = num_full * _ROWS_PER_W
                pltpu.sync_copy(idx_hbm.at[pl.ds(base, tail)], idx_t)
                pltpu.async_copy(table_hbm.at[idx_t], rows_t, sem).wait()
                pltpu.sync_copy(rows_t, out_hbm.at[pl.ds(base, tail)])

    return gather_kernel(table, idx)


def kernel(embedding_weight, seq_indices):
    rows = seq_indices.shape[0]
    idx = jnp.asarray(seq_indices, jnp.int32)
    return _sc_embedding_lookup(embedding_weight, idx, rows)


# speedup vs baseline: 1.1343x; 1.0622x over previous
import functools

import jax
import jax.numpy as jnp
from jax import lax
from jax.experimental import pallas as pl
from jax.experimental.pallas import tpu as pltpu
from jax.experimental.pallas import tpu_sc as plsc

_CHUNK = 4


@functools.partial(jax.jit, static_argnums=(2,))
def _sc_embedding_lookup(table, idx2d, rows):
    hidden = table.shape[1]
    num_w = idx2d.shape[0]
    mesh = plsc.VectorSubcoreMesh(
        core_axis_name="c", subcore_axis_name="s", num_cores=1
    )

    @functools.partial(
        pl.kernel,
        mesh=mesh,
        out_type=jax.ShapeDtypeStruct((rows, hidden), jnp.float32),
        scratch_types=[
            pltpu.VMEM((_CHUNK,), jnp.int32),
            pltpu.VMEM((_CHUNK, hidden), jnp.float32),
            pltpu.SemaphoreType.DMA,
        ],
    )
    def gather_kernel(table_hbm, idx_hbm, out_hbm, idx_v, rows_v, sem):
        wid = lax.axis_index("s")

        @pl.when(wid < num_w)
        def _():
            pltpu.sync_copy(idx_hbm.at[wid], idx_v)
            pltpu.async_copy(table_hbm.at[idx_v], rows_v, sem).wait()
            pltpu.sync_copy(rows_v, out_hbm.at[pl.ds(wid * _CHUNK, _CHUNK)])

    return gather_kernel(table, idx2d)


def kernel(embedding_weight, seq_indices):
    rows = seq_indices.shape[0]
    idx2d = jnp.asarray(seq_indices, jnp.int32).reshape(-1, _CHUNK)
    return _sc_embedding_lookup(embedding_weight, idx2d, rows)
